# baseline (speedup 1.0000x reference)
import jax
import jax.numpy as jnp
from jax import lax
from jax.experimental import pallas as pl
from jax.experimental.pallas import tpu as pltpu
from jax.experimental.pallas import tpu_sc as plsc

NH = 32
S = 513
NB = 16
WSTRIDE = 520
NW = 32
NTASK = 2 * S
TSLOT = 34
MHI = -65536


def _sc_body(ab_hbm, sp_hbm, wt_hbm, out_hbm,
             wcols, ab0, ab1, sp0, sp1, ix0, ix1, rb0, rb1, rb2, rb3,
             insem0, insem1, rsem0, rsem1, rsem2, rsem3):
    wid = lax.axis_index("s") * 2 + lax.axis_index("c")
    pltpu.sync_copy(wt_hbm, wcols)
    lane = lax.iota(jnp.int32, 16)
    c512 = jnp.full((16,), 512, jnp.int32)
    c511 = jnp.full((16,), 511, jnp.int32)
    rbufs = (rb0, rb1, rb2, rb3)
    rsems = (rsem0, rsem1, rsem2, rsem3)

    def in_copies(t, pb, build=False):
        tid = wid + t * NW
        i = tid // 2
        g8 = (tid & 1) * 8
        ab_b = ab0 if pb == 0 else ab1
        sp_b = sp0 if pb == 0 else sp1
        ix_b = ix0 if pb == 0 else ix1
        sem = insem0 if pb == 0 else insem1
        if build:
            base = jnp.maximum(g8 * 512 + i - 1, 0)
            ix_b[...] = jnp.minimum(base + lane * 512, NB * 512 - 1)
        return (
            pltpu.make_async_copy(ab_hbm.at[i, pl.ds(g8, 8), :], ab_b, sem),
            pltpu.make_async_copy(sp_hbm.at[ix_b], sp_b, sem),
        )

    def row_copy(t, nl):
        tid = wid + t * NW
        i = tid // 2
        n = (tid & 1) * 8 + nl
        q = nl % 4
        return pltpu.make_async_copy(rbufs[q], out_hbm.at[n, i], rsems[q])

    def task(t, pb):
        tid = wid + t * NW

        @pl.when(tid < NTASK)
        def _():
            @pl.when(tid + NW < NTASK)
            def _():
                for c in in_copies(t + 1, 1 - pb, build=True):
                    c.start()
            for c in in_copies(t, pb):
                c.wait()
            ab_b = ab0 if pb == 0 else ab1
            sp_b = sp0 if pb == 0 else sp1
            iz = (tid // 2) == 0
            for nl in range(8):
                rb = rbufs[nl % 4]
                if nl < 4:
                    @pl.when(t > 0)
                    def _(nl=nl):
                        row_copy(t - 1, nl + 4).wait()
                else:
                    row_copy(t, nl - 4).wait()

                def vec_body(jv, c, nl=nl, rb=rb):
                    off = jv * 16
                    p = off - 1 + lane
                    raw = plsc.load_gather(
                        sp_b, [jnp.full((16,), nl, jnp.int32),
                               jnp.maximum(p, 0)])
                    idx = jnp.where(jnp.logical_or(p < 0, iz), 0, raw)
                    ab = ab_b[nl, pl.ds(off, 16)]
                    ab2 = ab + ab
                    gs = [plsc.load_gather(wcols, [idx + hp * WSTRIDE])
                          for hp in range(NH // 2)]
                    for hp in range(NH // 2):
                        g = gs[hp]
                        f0 = plsc.bitcast(g << 16, jnp.float32)
                        f1 = plsc.bitcast(g & MHI, jnp.float32)
                        rb[2 * hp, pl.ds(off, 16)] = ab2 + f0
                        rb[2 * hp + 1, pl.ds(off, 16)] = ab2 + f1
                    return c

                lax.fori_loop(0, 32, vec_body, 0)
                nsp = jnp.full((16,), nl, jnp.int32)
                raw512 = plsc.load_gather(sp_b, [nsp, c511])
                idx512 = jnp.where(iz, 0, raw512)
                ab512 = plsc.load_gather(ab_b, [nsp, c512])
                ab2t = ab512 + ab512
                g = plsc.load_gather(wcols, [idx512 + lane * WSTRIDE])
                f0 = plsc.bitcast(g << 16, jnp.float32)
                f1 = plsc.bitcast(g & MHI, jnp.float32)
                plsc.store_scatter(rb, [2 * lane, c512], ab2t + f0)
                plsc.store_scatter(rb, [2 * lane + 1, c512], ab2t + f1)
                row_copy(t, nl).start()

    for c in in_copies(0, 0, build=True):
        c.start()

    def pair(tp, carry):
        task(2 * tp, 0)
        task(2 * tp + 1, 1)
        return carry

    lax.fori_loop(0, TSLOT // 2, pair, 0)

    last_t = jnp.where(wid < NTASK - NW * (TSLOT - 2), TSLOT - 2, TSLOT - 3)
    for nl in range(4, 8):
        row_copy(last_t, nl).wait()


@jax.jit
def _sc_call(ab_t, sp_t, wt):
    mesh = plsc.VectorSubcoreMesh(core_axis_name="c", subcore_axis_name="s")
    f = pl.kernel(
        _sc_body,
        out_type=jax.ShapeDtypeStruct((NB, S, NH, S), jnp.float32),
        mesh=mesh,
        compiler_params=pltpu.CompilerParams(needs_layout_passes=False,
                                             use_tc_tiling_on_sc=True),
        scratch_types=[
            pltpu.VMEM((NH // 2 * WSTRIDE,), jnp.int32),
            pltpu.VMEM((8, S), jnp.float32),
            pltpu.VMEM((8, S), jnp.float32),
            pltpu.VMEM((16, 512), jnp.int32),
            pltpu.VMEM((16, 512), jnp.int32),
            pltpu.VMEM((16,), jnp.int32),
            pltpu.VMEM((16,), jnp.int32),
            pltpu.VMEM((NH, S), jnp.float32),
            pltpu.VMEM((NH, S), jnp.float32),
            pltpu.VMEM((NH, S), jnp.float32),
            pltpu.VMEM((NH, S), jnp.float32),
            pltpu.SemaphoreType.DMA,
            pltpu.SemaphoreType.DMA,
            pltpu.SemaphoreType.DMA,
            pltpu.SemaphoreType.DMA,
            pltpu.SemaphoreType.DMA,
            pltpu.SemaphoreType.DMA,
        ],
    )
    return f(ab_t, sp_t, wt)


def kernel(attn_bias, spatial_pos, x, edge_input, attn_edge_type, spatial_W):
    del x, edge_input, attn_edge_type
    W0 = spatial_W.at[0].set(0.0)
    Wu = lax.bitcast_convert_type(W0.astype(jnp.bfloat16),
                                  jnp.uint16).astype(jnp.uint32)
    pair = Wu[:, 0::2] | (Wu[:, 1::2] << 16)
    wt = lax.bitcast_convert_type(pair, jnp.int32).T
    wt = jnp.pad(wt, ((0, 0), (0, WSTRIDE - S))).reshape(-1)
    ab_t = jnp.transpose(attn_bias, (1, 0, 2))
    sp2 = spatial_pos.reshape(NB * 512, 512)
    out5 = _sc_call(ab_t, sp2, wt)
    return jnp.transpose(out5, (0, 2, 1, 3))


# speedup vs baseline: 1.9813x; 1.9813x over previous
import jax
import jax.numpy as jnp
from jax import lax
from jax.experimental import pallas as pl
from jax.experimental.pallas import tpu as pltpu
from jax.experimental.pallas import tpu_sc as plsc

NH = 32
S = 513
NB = 16
WSTRIDE = 520
NW = 32
NTASK = 2 * S
TSLOT = 34
MHI = -65536


def _sc_body(ab_hbm, sp_hbm, wt_hbm, out_hbm,
             wcols, ab0, ab1, sp0, sp1, rb0, rb1, rb2, rb3,
             insem0, insem1, rsem0, rsem1, rsem2, rsem3):
    wid = lax.axis_index("s") * 2 + lax.axis_index("c")
    pltpu.sync_copy(wt_hbm, wcols)
    lane = lax.iota(jnp.int32, 16)
    c512 = jnp.full((16,), 512, jnp.int32)
    rbufs = (rb0, rb1, rb2, rb3)
    rsems = (rsem0, rsem1, rsem2, rsem3)

    def in_copies(t, pb):
        tid = wid + t * NW
        i = tid // 2
        g8 = (tid & 1) * 8
        ab_b = ab0 if pb == 0 else ab1
        sp_b = sp0 if pb == 0 else sp1
        sem = insem0 if pb == 0 else insem1
        return (
            pltpu.make_async_copy(ab_hbm.at[i, pl.ds(g8, 8), :], ab_b, sem),
            pltpu.make_async_copy(sp_hbm.at[i, pl.ds(g8, 8), :], sp_b, sem),
        )

    def row_copy(t, nl):
        tid = wid + t * NW
        i = tid // 2
        n = (tid & 1) * 8 + nl
        q = nl % 4
        return pltpu.make_async_copy(rbufs[q], out_hbm.at[n, i], rsems[q])

    def task(t, pb):
        tid = wid + t * NW

        @pl.when(tid < NTASK)
        def _():
            @pl.when(tid + NW < NTASK)
            def _():
                for c in in_copies(t + 1, 1 - pb):
                    c.start()
            for c in in_copies(t, pb):
                c.wait()
            ab_b = ab0 if pb == 0 else ab1
            sp_b = sp0 if pb == 0 else sp1
            for nl in range(8):
                rb = rbufs[nl % 4]
                if nl < 4:
                    @pl.when(t > 0)
                    def _(nl=nl):
                        row_copy(t - 1, nl + 4).wait()
                else:
                    row_copy(t, nl - 4).wait()

                def vec_body(jv, c, nl=nl, rb=rb):
                    off = jv * 16
                    idx = sp_b[nl, pl.ds(off, 16)]
                    ab = ab_b[nl, pl.ds(off, 16)]
                    ab2 = ab + ab
                    gs = [plsc.load_gather(wcols, [idx + hp * WSTRIDE])
                          for hp in range(NH // 2)]
                    for hp in range(NH // 2):
                        g = gs[hp]
                        f0 = plsc.bitcast(g << 16, jnp.float32)
                        f1 = plsc.bitcast(g & MHI, jnp.float32)
                        rb[2 * hp, pl.ds(off, 16)] = ab2 + f0
                        rb[2 * hp + 1, pl.ds(off, 16)] = ab2 + f1
                    return c

                lax.fori_loop(0, 32, vec_body, 0)
                nsp = jnp.full((16,), nl, jnp.int32)
                idx512 = plsc.load_gather(sp_b, [nsp, c512])
                ab512 = plsc.load_gather(ab_b, [nsp, c512])
                ab2t = ab512 + ab512
                g = plsc.load_gather(wcols, [idx512 + lane * WSTRIDE])
                f0 = plsc.bitcast(g << 16, jnp.float32)
                f1 = plsc.bitcast(g & MHI, jnp.float32)
                plsc.store_scatter(rb, [2 * lane, c512], ab2t + f0)
                plsc.store_scatter(rb, [2 * lane + 1, c512], ab2t + f1)
                row_copy(t, nl).start()

    for c in in_copies(0, 0):
        c.start()

    def pair(tp, carry):
        task(2 * tp, 0)
        task(2 * tp + 1, 1)
        return carry

    lax.fori_loop(0, TSLOT // 2, pair, 0)

    last_t = jnp.where(wid < NTASK - NW * (TSLOT - 2), TSLOT - 2, TSLOT - 3)
    for nl in range(4, 8):
        row_copy(last_t, nl).wait()


@jax.jit
def _sc_call(ab_t, sp_t, wt):
    mesh = plsc.VectorSubcoreMesh(core_axis_name="c", subcore_axis_name="s")
    f = pl.kernel(
        _sc_body,
        out_type=jax.ShapeDtypeStruct((NB, S, NH, S), jnp.float32),
        mesh=mesh,
        compiler_params=pltpu.CompilerParams(needs_layout_passes=False,
                                             use_tc_tiling_on_sc=True),
        scratch_types=[
            pltpu.VMEM((NH // 2 * WSTRIDE,), jnp.int32),
            pltpu.VMEM((8, S), jnp.float32),
            pltpu.VMEM((8, S), jnp.float32),
            pltpu.VMEM((8, S), jnp.int32),
            pltpu.VMEM((8, S), jnp.int32),
            pltpu.VMEM((NH, S), jnp.float32),
            pltpu.VMEM((NH, S), jnp.float32),
            pltpu.VMEM((NH, S), jnp.float32),
            pltpu.VMEM((NH, S), jnp.float32),
            pltpu.SemaphoreType.DMA,
            pltpu.SemaphoreType.DMA,
            pltpu.SemaphoreType.DMA,
            pltpu.SemaphoreType.DMA,
            pltpu.SemaphoreType.DMA,
            pltpu.SemaphoreType.DMA,
        ],
    )
    return f(ab_t, sp_t, wt)


def kernel(attn_bias, spatial_pos, x, edge_input, attn_edge_type, spatial_W):
    del x, edge_input, attn_edge_type
    W0 = spatial_W.at[0].set(0.0)
    Wu = lax.bitcast_convert_type(W0.astype(jnp.bfloat16),
                                  jnp.uint16).astype(jnp.uint32)
    pair = Wu[:, 0::2] | (Wu[:, 1::2] << 16)
    wt = lax.bitcast_convert_type(pair, jnp.int32).T
    wt = jnp.pad(wt, ((0, 0), (0, WSTRIDE - S))).reshape(-1)
    ab_t = jnp.transpose(attn_bias, (1, 0, 2))
    sp_pad = jnp.pad(spatial_pos, ((0, 0), (1, 0), (1, 0)))
    sp_t = jnp.transpose(sp_pad, (1, 0, 2))
    out5 = _sc_call(ab_t, sp_t, wt)
    return jnp.transpose(out5, (0, 2, 1, 3))
